# baseline (speedup 1.0000x reference)
import jax
import jax.numpy as jnp
from jax import lax
from jax.experimental import pallas as pl
from jax.experimental.pallas import tpu as pltpu, tpu_sc as plsc

NUM_CLASSES = 150
H = 512
W = 512
NPIX = H * W
NC = 2
NS = 16
NWORKERS = NC * NS
PIX_PER_WORKER = NPIX // NWORKERS
CHUNK = 512
NCHUNKS = PIX_PER_WORKER // CHUNK
L = 16


def _one_hot_body(x_hbm, out_hbm, labels_v, buf_v):
    cid = lax.axis_index("c")
    sid = lax.axis_index("s")
    wid = sid * NC + cid
    base_px = wid * PIX_PER_WORKER

    def zero_body(t, carry):
        buf_v[t // (CHUNK // L), pl.ds((t % (CHUNK // L)) * L, L)] = (
            jnp.zeros((L,), jnp.int32))
        return carry
    lax.fori_loop(0, NUM_CLASSES * (CHUNK // L), zero_body, 0)

    pltpu.sync_copy(x_hbm.at[pl.ds(base_px, PIX_PER_WORKER)], labels_v)

    ones = jnp.ones((L,), jnp.int32)
    zeros = jnp.zeros((L,), jnp.int32)
    lane = lax.iota(jnp.int32, L)

    def chunk_body(chunk, carry):
        off = chunk * CHUNK
        for k in range(CHUNK // L):
            lab = labels_v[pl.ds(off + k * L, L)]
            pos = lane + (k * L)
            plsc.store_scatter(buf_v, [lab, pos], ones)
        pltpu.sync_copy(
            buf_v, out_hbm.at[:, pl.ds(base_px + off, CHUNK)])
        for k in range(CHUNK // L):
            lab = labels_v[pl.ds(off + k * L, L)]
            pos = lane + (k * L)
            plsc.store_scatter(buf_v, [lab, pos], zeros)
        return carry
    lax.fori_loop(0, NCHUNKS, chunk_body, 0)


@jax.jit
def _one_hot(x):
    k = pl.kernel(
        _one_hot_body,
        out_type=jax.ShapeDtypeStruct((NUM_CLASSES, NPIX), jnp.int32),
        mesh=plsc.VectorSubcoreMesh(core_axis_name="c", subcore_axis_name="s"),
        scratch_types=[
            pltpu.VMEM((PIX_PER_WORKER,), jnp.int32),
            pltpu.VMEM((NUM_CLASSES, CHUNK), jnp.int32),
        ],
        compiler_params=pltpu.CompilerParams(
            use_tc_tiling_on_sc=False, needs_layout_passes=False),
    )
    return k(x)


def kernel(img):
    x = img.astype(jnp.int32).reshape(NPIX)
    out = _one_hot(x)
    return out.reshape(NUM_CLASSES, H, W)
